# baseline (speedup 1.0000x reference)
import functools

import jax
import jax.numpy as jnp
from jax import lax
from jax.experimental import pallas as pl
from jax.experimental.pallas import tpu as pltpu
from jax.experimental.pallas import tpu_sc as plsc

VOCAB = 10
DIM = 10
TOK = 200
ROWS = 16384
N = TOK * ROWS
NW = 32
PW = N // NW
K = 3200
NCH = PW // K


def _sc_body(x_hbm, w1_hbm, w2_hbm, out_hbm, x_v, w1_v, w2_v, lut_v, out_v,
             sem):
    wid = lax.axis_index("s") * 2 + lax.axis_index("c")
    base = wid * PW

    pltpu.sync_copy(w1_hbm, w1_v)
    pltpu.sync_copy(w2_hbm, w2_v)
    vi = lax.iota(jnp.int32, 16)
    msk = vi < VOCAB
    vc = jnp.where(msk, vi, 0)
    for d in range(DIM):
        addr = vc * DIM + d
        a = plsc.load_gather(w1_v, [addr], mask=msk)
        b = plsc.load_gather(w2_v, [addr], mask=msk)
        lut_v[pl.ds(d * 16, 16)] = a + b

    pending = [[], []]
    for it in range(NCH):
        off = base + it * K
        b = it % 2
        pltpu.sync_copy(x_hbm.at[pl.ds(off, K)], x_v)
        for cp in pending[b]:
            cp.wait()
        pending[b] = []

        def body(j, _):
            xv = x_v[pl.ds(j * 16, 16)]
            for d in range(DIM):
                addr = xv + (16 * d)
                val = plsc.load_gather(lut_v, [addr])
                out_v[b, d, pl.ds(j * 16, 16)] = val
            return 0

        lax.fori_loop(0, K // 16, body, 0)
        for d in range(DIM):
            cp = pltpu.make_async_copy(
                out_v.at[b, d], out_hbm.at[d, pl.ds(off, K)], sem.at[b])
            cp.start()
            pending[b].append(cp)
    for b in range(2):
        for cp in pending[b]:
            cp.wait()


@jax.jit
def kernel(x, W1, W2):
    xflat = x.T.reshape(N)
    mesh = plsc.VectorSubcoreMesh(core_axis_name="c", subcore_axis_name="s")
    outflat = pl.kernel(
        _sc_body,
        mesh=mesh,
        compiler_params=pltpu.CompilerParams(needs_layout_passes=False),
        out_type=jax.ShapeDtypeStruct((DIM, N), jnp.float32),
        scratch_types=[
            pltpu.VMEM((K,), jnp.int32),
            pltpu.VMEM((VOCAB * DIM,), jnp.float32),
            pltpu.VMEM((VOCAB * DIM,), jnp.float32),
            pltpu.VMEM((16 * DIM,), jnp.float32),
            pltpu.VMEM((2, DIM, K), jnp.float32),
            pltpu.SemaphoreType.DMA((2,)),
        ],
    )(xflat, W1.reshape(VOCAB * DIM), W2.reshape(VOCAB * DIM))
    return outflat.reshape(DIM, TOK, ROWS).transpose(2, 1, 0)
